# Initial kernel scaffold; baseline (speedup 1.0000x reference)
import jax
import jax.numpy as jnp
from jax.experimental import pallas as pl


def kernel(morph_ids, feature, Source_num, morph_table, W_f, b_f, cls_token, src_table):
    raise NotImplementedError("write your pallas kernel here")


# speedup vs baseline: 1.0379x; 1.0379x over previous
import functools

import jax
import jax.numpy as jnp
from jax import lax
from jax.experimental import pallas as pl
from jax.experimental.pallas import tpu as pltpu
from jax.experimental.pallas import tpu_sc as plsc


def kernel(morph_ids, feature, Source_num, morph_table, W_f, b_f, cls_token, src_table):
    B, Lp1 = morph_ids.shape
    V, D = morph_table.shape
    T = Lp1 + 1
    P = 208
    NJ = D // 16
    HALF = P // 2

    ids_p = jnp.pad(morph_ids.astype(jnp.int32), ((0, 0), (0, P - Lp1)))
    ids_p = ids_p.reshape(B, 2, HALF)
    feat_p = jnp.pad(feature, ((0, 0), (0, P - feature.shape[1])))
    wf_r = W_f.reshape(D)
    cls_r = cls_token.reshape(D)
    src_r = src_table.reshape(-1)
    sn_r = Source_num.astype(jnp.int32)

    info = plsc.get_sparse_core_info()
    NW = info.num_cores * info.num_subcores
    NB = B // NW
    NC = info.num_cores

    mesh = plsc.VectorSubcoreMesh(core_axis_name="c", subcore_axis_name="s")

    @functools.partial(
        pl.kernel,
        out_type=jax.ShapeDtypeStruct((B, T, D), jnp.float32),
        mesh=mesh,
        compiler_params=pltpu.CompilerParams(
            needs_layout_passes=False, use_tc_tiling_on_sc=False),
        scratch_types=[
            pltpu.VMEM((2, 2, HALF), jnp.int32),
            pltpu.VMEM((P,), jnp.float32),
            pltpu.VMEM((P,), jnp.float32),
            pltpu.VMEM((2, P, D), jnp.float32),
            pltpu.VMEM((2, P, D), jnp.float32),
            pltpu.VMEM((NB,), jnp.int32),
            pltpu.VMEM((D,), jnp.float32),
            pltpu.VMEM((D,), jnp.float32),
            pltpu.VMEM((D,), jnp.float32),
            pltpu.VMEM((src_r.shape[0],), jnp.float32),
            pltpu.SemaphoreType.DMA,
            pltpu.SemaphoreType.DMA,
            pltpu.SemaphoreType.DMA,
            pltpu.SemaphoreType.DMA,
            pltpu.SemaphoreType.DMA,
            pltpu.SemaphoreType.DMA,
        ],
    )
    def run(ids_h, feat_h, sn_h, tab_h, wf_h, bf_h, cls_h, src_h, out_h,
            idx2, feA, feB, G2, O2, sn_v, wf_v, bf_v, cls_v, src_v,
            isem0, isem1, gsem0, gsem1, wsem0, wsem1):
        wid = lax.axis_index("s") * NC + lax.axis_index("c")
        base = wid * NB
        isems = (isem0, isem1)
        gsems = (gsem0, gsem1)
        wsems = (wsem0, wsem1)

        pltpu.sync_copy(wf_h, wf_v)
        pltpu.sync_copy(bf_h, bf_v)
        pltpu.sync_copy(cls_h, cls_v)
        pltpu.sync_copy(src_h, src_v)
        pltpu.sync_copy(sn_h.at[pl.ds(base, NB)], sn_v)

        wvs = [wf_v[pl.ds(16 * j, 16)] for j in range(NJ)]
        bvs = [bf_v[pl.ds(16 * j, 16)] for j in range(NJ)]
        cvs = [cls_v[pl.ds(16 * j, 16)] for j in range(NJ)]

        fe_bufs = (feA, feB)

        def in_copies(s, bg):
            return (pltpu.make_async_copy(ids_h.at[bg], idx2.at[s], isems[s]),
                    pltpu.make_async_copy(feat_h.at[bg], fe_bufs[s], isems[s]))

        def gather_copies(s):
            idxs = idx2.at[s]
            return tuple(
                pltpu.make_async_copy(
                    tab_h.at[idxs.at[h]],
                    G2.at[s].at[pl.ds(h * HALF, HALF)],
                    gsems[s])
                for h in range(2))

        def out_copy(s, bg):
            return pltpu.make_async_copy(O2.at[s].at[pl.ds(0, T)], out_h.at[bg], wsems[s])

        def start(copies):
            for c in copies:
                c.start()

        def wait(copies):
            for c in copies:
                c.wait()

        def compute(s, bl):
            Gs, Os, fes = G2.at[s], O2.at[s], fe_bufs[s]
            for j in range(NJ):
                Os[0, pl.ds(16 * j, 16)] = Gs[0, pl.ds(16 * j, 16)] + cvs[j]

            @pl.loop(1, Lp1, unroll=4)
            def _(l):
                fv = plsc.load_gather(fes, [jnp.full((16,), l - 1, jnp.int32)])
                for j in range(NJ):
                    Os[l, pl.ds(16 * j, 16)] = (
                        Gs[l, pl.ds(16 * j, 16)] + (fv * wvs[j] + bvs[j]))

            sbase = plsc.load_gather(sn_v, [jnp.full((16,), bl, jnp.int32)]) * D
            for j in range(NJ):
                idxv = sbase + 16 * j + lax.iota(jnp.int32, 16)
                Os[Lp1, pl.ds(16 * j, 16)] = plsc.load_gather(src_v, [idxv])

        start(in_copies(0, base))
        wait(in_copies(0, base))
        start(gather_copies(0))
        start(in_copies(1, base + 1))

        @pl.loop(0, NB, step=2)
        def _(b0):
            for s in (0, 1):
                bl = b0 + s
                bg = base + bl
                so = s ^ 1

                @pl.when(bl + 1 < NB)
                def _():
                    wait(in_copies(so, bg + 1))
                    start(gather_copies(so))

                wait(gather_copies(s))

                @pl.when(bl >= 2)
                def _():
                    out_copy(s, bg - 2).wait()

                compute(s, bl)
                out_copy(s, bg).start()

                @pl.when(bl + 2 < NB)
                def _():
                    start(in_copies(s, bg + 2))

        out_copy(0, base + NB - 2).wait()
        out_copy(1, base + NB - 1).wait()

    return run(ids_p, feat_p, sn_r, morph_table, wf_r, b_f, cls_r, src_r)


# speedup vs baseline: 1.0435x; 1.0054x over previous
import functools

import jax
import jax.numpy as jnp
from jax import lax
from jax.experimental import pallas as pl
from jax.experimental.pallas import tpu as pltpu
from jax.experimental.pallas import tpu_sc as plsc

_R = 4


def kernel(morph_ids, feature, Source_num, morph_table, W_f, b_f, cls_token, src_table):
    B, Lp1 = morph_ids.shape
    V, D = morph_table.shape
    T = Lp1 + 1
    P = 208
    NJ = D // 16
    HALF = P // 2

    ids_p = jnp.pad(morph_ids.astype(jnp.int32), ((0, 0), (0, P - Lp1)))
    ids_p = ids_p.reshape(B, 2, HALF)
    feat_p = jnp.pad(feature, ((0, 0), (0, P - feature.shape[1])))
    wf_r = W_f.reshape(D)
    cls_r = cls_token.reshape(D)
    src_r = src_table.reshape(-1)
    sn_r = Source_num.astype(jnp.int32)

    info = plsc.get_sparse_core_info()
    NW = info.num_cores * info.num_subcores
    NB = B // NW
    NC = info.num_cores

    mesh = plsc.VectorSubcoreMesh(core_axis_name="c", subcore_axis_name="s")

    @functools.partial(
        pl.kernel,
        out_type=jax.ShapeDtypeStruct((B, T, D), jnp.float32),
        mesh=mesh,
        compiler_params=pltpu.CompilerParams(
            needs_layout_passes=False, use_tc_tiling_on_sc=False),
        scratch_types=[
            pltpu.VMEM((_R, 2, HALF), jnp.int32),
            [pltpu.VMEM((P,), jnp.float32) for _ in range(_R)],
            pltpu.VMEM((_R, P, D), jnp.float32),
            pltpu.VMEM((NB,), jnp.int32),
            pltpu.VMEM((D,), jnp.float32),
            pltpu.VMEM((D,), jnp.float32),
            pltpu.VMEM((D,), jnp.float32),
            pltpu.VMEM((src_r.shape[0],), jnp.float32),
            [pltpu.SemaphoreType.DMA for _ in range(_R)],
            [pltpu.SemaphoreType.DMA for _ in range(_R)],
            [pltpu.SemaphoreType.DMA for _ in range(_R)],
        ],
    )
    def run(ids_h, feat_h, sn_h, tab_h, wf_h, bf_h, cls_h, src_h, out_h,
            idx, fe_bufs, G, sn_v, wf_v, bf_v, cls_v, src_v,
            isems, gsems, wsems):
        wid = lax.axis_index("s") * NC + lax.axis_index("c")
        base = wid * NB

        pltpu.sync_copy(wf_h, wf_v)
        pltpu.sync_copy(bf_h, bf_v)
        pltpu.sync_copy(cls_h, cls_v)
        pltpu.sync_copy(src_h, src_v)
        pltpu.sync_copy(sn_h.at[pl.ds(base, NB)], sn_v)

        wvs = [wf_v[pl.ds(16 * j, 16)] for j in range(NJ)]
        bvs = [bf_v[pl.ds(16 * j, 16)] for j in range(NJ)]
        cvs = [cls_v[pl.ds(16 * j, 16)] for j in range(NJ)]

        def in_copies(s, bg):
            return (pltpu.make_async_copy(ids_h.at[bg], idx.at[s], isems[s]),
                    pltpu.make_async_copy(feat_h.at[bg], fe_bufs[s], isems[s]))

        def start_gathers(s):
            for h in range(2):
                pltpu.async_copy(
                    tab_h.at[idx.at[s].at[h]],
                    G.at[s].at[pl.ds(h * HALF, HALF)],
                    gsems[s], add=True)

        def wait_gathers(s):
            for h in range(2):
                pltpu.make_async_copy(
                    tab_h.at[idx.at[s].at[h]],
                    G.at[s].at[pl.ds(h * HALF, HALF)],
                    gsems[s]).wait()

        def out_copy(s, bg):
            return pltpu.make_async_copy(G.at[s].at[pl.ds(0, T)], out_h.at[bg], wsems[s])

        def start(copies):
            for c in copies:
                c.start()

        def wait(copies):
            for c in copies:
                c.wait()

        def fe_compute(s):
            Gs, fes = G.at[s], fe_bufs[s]
            for j in range(NJ):
                Gs[0, pl.ds(16 * j, 16)] = cvs[j]

            @pl.loop(1, Lp1, unroll=4)
            def _(l):
                fv = plsc.load_gather(fes, [jnp.full((16,), l - 1, jnp.int32)])
                for j in range(NJ):
                    Gs[l, pl.ds(16 * j, 16)] = fv * wvs[j] + bvs[j]

        def src_fix(s, bl):
            Gs = G.at[s]
            sbase = plsc.load_gather(sn_v, [jnp.full((16,), bl, jnp.int32)]) * D
            for j in range(NJ):
                idxv = sbase + 16 * j + lax.iota(jnp.int32, 16)
                Gs[Lp1, pl.ds(16 * j, 16)] = plsc.load_gather(src_v, [idxv])

        for i in range(_R):
            start(in_copies(i, base + i))
        for i in range(_R - 1):
            wait(in_copies(i, base + i))
            fe_compute(i)
            start_gathers(i)

        @pl.loop(0, NB, step=_R)
        def _(b0):
            for s in range(_R):
                bl = b0 + s
                bg = base + bl

                wait_gathers(s)
                src_fix(s, bl)
                out_copy(s, bg).start()

                @pl.when(bl + _R - 1 < NB)
                def _():
                    sn = (s + _R - 1) % _R
                    wait(in_copies(sn, bg + _R - 1))

                    @pl.when(bl >= 1)
                    def _():
                        out_copy(sn, bg - 1).wait()

                    fe_compute(sn)
                    start_gathers(sn)

                @pl.when(bl + _R < NB)
                def _():
                    start(in_copies(s, bg + _R))

        for i in range(_R):
            out_copy(i, base + NB - _R + i).wait()

    return run(ids_p, feat_p, sn_r, morph_table, wf_r, b_f, cls_r, src_r)
